# baseline (speedup 1.0000x reference)
import functools

import jax
import jax.numpy as jnp
from jax import lax
from jax.experimental import pallas as pl
from jax.experimental.pallas import tpu as pltpu
from jax.experimental.pallas import tpu_sc as plsc

_BATCH_BLOCK = 32
_TC_ROWS = 672


def _broadcast_body(pos_emb_ref, out_ref):
    out_ref[...] = jnp.broadcast_to(pos_emb_ref[...][None], out_ref.shape)


def _tc_part(pos_emb, rows):
    seq, dim = pos_emb.shape
    return pl.pallas_call(
        _broadcast_body,
        grid=(rows // _BATCH_BLOCK,),
        in_specs=[pl.BlockSpec((seq, dim), lambda i: (0, 0))],
        out_specs=pl.BlockSpec((_BATCH_BLOCK, seq, dim), lambda i: (i, 0, 0)),
        out_shape=jax.ShapeDtypeStruct((rows, seq, dim), jnp.float32),
    )(pos_emb)


def _sc_part(pos_emb, rows):
    seq, dim = pos_emb.shape
    info = plsc.get_sparse_core_info()
    nworkers = info.num_cores * info.num_subcores
    b_per_w = rows // nworkers
    mesh = plsc.VectorSubcoreMesh(core_axis_name="c", subcore_axis_name="s")

    @functools.partial(
        pl.kernel,
        mesh=mesh,
        out_type=jax.ShapeDtypeStruct((rows, seq, dim), jnp.float32),
        scratch_types=[
            pltpu.VMEM((seq, dim), jnp.float32),
            pltpu.SemaphoreType.DMA,
        ],
    )
    def k(pos_hbm, out_hbm, tab_v, sem):
        wid = lax.axis_index("s") * info.num_cores + lax.axis_index("c")
        base = wid * b_per_w
        pltpu.sync_copy(pos_hbm, tab_v)
        for b in range(b_per_w):
            pltpu.make_async_copy(tab_v, out_hbm.at[base + b], sem).start()
        for b in range(b_per_w):
            pltpu.make_async_copy(tab_v, out_hbm.at[base + b], sem).wait()

    return k(pos_emb)


def kernel(x, pos_emb):
    batch = x.shape[0]
    tc = _tc_part(pos_emb, _TC_ROWS)
    sc = _sc_part(pos_emb, batch - _TC_ROWS)
    return jnp.concatenate([tc, sc], axis=0)


# speedup vs baseline: 3.8074x; 3.8074x over previous
import jax
import jax.numpy as jnp
from jax.experimental import pallas as pl

_BATCH_BLOCK = 32


def _broadcast_body(pos_emb_ref, out_ref):
    out_ref[...] = jnp.broadcast_to(pos_emb_ref[...][None], out_ref.shape)


def kernel(x, pos_emb):
    batch = x.shape[0]
    seq, dim = pos_emb.shape
    return pl.pallas_call(
        _broadcast_body,
        grid=(batch // _BATCH_BLOCK,),
        in_specs=[pl.BlockSpec((seq, dim), lambda i: (0, 0))],
        out_specs=pl.BlockSpec((_BATCH_BLOCK, seq, dim), lambda i: (i, 0, 0)),
        out_shape=jax.ShapeDtypeStruct((batch, seq, dim), jnp.float32),
    )(pos_emb)
